# baseline (speedup 1.0000x reference)
import functools

import jax
import jax.numpy as jnp
from jax import lax
from jax.experimental import pallas as pl
from jax.experimental.pallas import tpu as pltpu
from jax.experimental.pallas import tpu_sc as plsc

S, D, H, HD = 2048, 1024, 16, 64
E, FF, SFF = 8, 1408, 5632
EPS = 1e-6
F32 = jnp.float32

BR = 256
NQ = S // BR
BLK = 128
NB = (2 * S) // BLK + E
NPAD = NB * BLK
NB_LANE = 40
NF = SFF // FF
W = 32
SCALE = 1.0 / 8.0
NEG = -1e30


def _qkv_body(x_ref, ln_ref, wq_ref, bq_ref, wk_ref, bk_ref, wv_ref, bv_ref,
              cos_ref, sin_ref, q_ref, k_ref, v_ref):
    x = x_ref[...]
    var = jnp.mean(x * x, axis=1, keepdims=True)
    hn = (x * lax.rsqrt(var + EPS) * ln_ref[...]).astype(jnp.bfloat16)
    q = jnp.dot(hn, wq_ref[...].astype(jnp.bfloat16),
                preferred_element_type=F32) + bq_ref[...]
    k = jnp.dot(hn, wk_ref[...].astype(jnp.bfloat16),
                preferred_element_type=F32) + bk_ref[...]
    v = jnp.dot(hn, wv_ref[...].astype(jnp.bfloat16),
                preferred_element_type=F32) + bv_ref[...]
    cos = cos_ref[...]
    sin = sin_ref[...]
    half = D // 2
    qrot = jnp.concatenate([-q[:, half:], q[:, :half]], axis=1)
    krot = jnp.concatenate([-k[:, half:], k[:, :half]], axis=1)
    q_ref[...] = ((q * cos + qrot * sin) * SCALE).astype(jnp.bfloat16)
    k_ref[...] = (k * cos + krot * sin).astype(jnp.bfloat16)
    v_ref[...] = v.astype(jnp.bfloat16)


def _qkv_call(x, ln1, wq, bq, wk, bk, wv, bv, cos, sin):
    hsd = jax.ShapeDtypeStruct((S, D), jnp.bfloat16)
    return pl.pallas_call(
        _qkv_body,
        grid=(NQ,),
        in_specs=[
            pl.BlockSpec((BR, D), lambda i: (i, 0)),
            pl.BlockSpec((1, D), lambda i: (0, 0)),
            pl.BlockSpec((D, D), lambda i: (0, 0)),
            pl.BlockSpec((1, D), lambda i: (0, 0)),
            pl.BlockSpec((D, D), lambda i: (0, 0)),
            pl.BlockSpec((1, D), lambda i: (0, 0)),
            pl.BlockSpec((D, D), lambda i: (0, 0)),
            pl.BlockSpec((1, D), lambda i: (0, 0)),
            pl.BlockSpec((BR, D), lambda i: (i, 0)),
            pl.BlockSpec((BR, D), lambda i: (i, 0)),
        ],
        out_specs=[
            pl.BlockSpec((BR, D), lambda i: (i, 0)),
            pl.BlockSpec((BR, D), lambda i: (i, 0)),
            pl.BlockSpec((BR, D), lambda i: (i, 0)),
        ],
        out_shape=[hsd, hsd, hsd],
    )(x, ln1, wq, bq, wk, bk, wv, bv, cos, sin)


BQA = 512
NQA = S // BQA
TRI = NQA * (NQA + 1) // 2


def _attn_body(sq_ref, sk_ref, sd_ref, q_ref, k_ref, v_ref, mask_ref, o_ref,
               l_sc, a_sc):
    t = pl.program_id(1)
    iq = sq_ref[t]
    jk = sk_ref[t]

    @pl.when(jk == 0)
    def _():
        l_sc[...] = jnp.zeros((BQA, 128), F32)
        a_sc[...] = jnp.zeros((BQA, HD), F32)

    q = q_ref[0]
    kk = k_ref[0]
    s = lax.dot_general(q, kk, (((1,), (1,)), ((), ())),
                        preferred_element_type=F32)
    p = jnp.exp(jnp.minimum(s + mask_ref[0], 60.0))
    l_new = l_sc[:, 0:1] + jnp.sum(p, axis=1, keepdims=True)
    a_new = a_sc[...] + jnp.dot(
        p.astype(jnp.bfloat16), v_ref[0], preferred_element_type=F32)
    l_sc[...] = jnp.broadcast_to(l_new, (BQA, 128))
    a_sc[...] = a_new

    @pl.when(jk == iq)
    def _():
        o_ref[0] = a_new / l_new


def _attn_call(q, k, v):
    sq = []
    sk = []
    for i in range(NQA):
        for j in range(i + 1):
            sq.append(i)
            sk.append(j)
    diag = [int(a == b) for a, b in zip(sq, sk)]
    sq = jnp.asarray(sq, jnp.int32)
    sk = jnp.asarray(sk, jnp.int32)
    sd = jnp.asarray(diag, jnp.int32)
    r = jnp.arange(BQA, dtype=jnp.int32)
    causal = jnp.where(r[:, None] >= r[None, :], 0.0, NEG).astype(F32)
    masks = jnp.stack([jnp.zeros((BQA, BQA), F32), causal])
    grid_spec = pltpu.PrefetchScalarGridSpec(
        num_scalar_prefetch=3,
        grid=(H, TRI),
        in_specs=[
            pl.BlockSpec((1, BQA, HD), lambda h, t, sq, sk, sd: (h, sq[t], 0)),
            pl.BlockSpec((1, BQA, HD), lambda h, t, sq, sk, sd: (h, sk[t], 0)),
            pl.BlockSpec((1, BQA, HD), lambda h, t, sq, sk, sd: (h, sk[t], 0)),
            pl.BlockSpec((1, BQA, BQA), lambda h, t, sq, sk, sd: (sd[t], 0, 0)),
        ],
        out_specs=pl.BlockSpec(
            (1, BQA, HD), lambda h, t, sq, sk, sd: (h, sq[t], 0)),
        scratch_shapes=[
            pltpu.VMEM((BQA, 128), F32),
            pltpu.VMEM((BQA, HD), F32),
        ],
    )
    return pl.pallas_call(
        _attn_body,
        grid_spec=grid_spec,
        out_shape=jax.ShapeDtypeStruct((H, S, HD), F32),
    )(sq, sk, sd, q, k, v, masks)


def _post_body(o_ref, x_ref, wo_ref, ln2_ref, wr_ref, wsh_ref,
               x2_ref, h2_ref, tw_ref, ti_ref, gate_ref):
    o = o_ref[...]
    x2 = x_ref[...] + jnp.dot(o.astype(jnp.bfloat16),
                              wo_ref[...].astype(jnp.bfloat16),
                              preferred_element_type=F32)
    x2_ref[...] = x2
    var = jnp.mean(x2 * x2, axis=1, keepdims=True)
    h2 = x2 * lax.rsqrt(var + EPS) * ln2_ref[...]
    h2_ref[...] = h2
    lg = jnp.dot(h2, wsh_ref[...], preferred_element_type=F32)
    gate_ref[...] = 1.0 / (1.0 + jnp.exp(-lg))
    logits = jnp.dot(h2, wr_ref[...], preferred_element_type=F32)
    lane = lax.broadcasted_iota(jnp.int32, (BR, 128), 1)
    logits = jnp.where(lane < E, logits, NEG)
    mx = jnp.max(logits, axis=1, keepdims=True)
    ex = jnp.exp(logits - mx)
    probs = ex / jnp.sum(ex, axis=1, keepdims=True)
    w1 = jnp.max(probs, axis=1, keepdims=True)
    i1 = jnp.min(jnp.where(probs == w1, lane, 999), axis=1, keepdims=True)
    p2 = jnp.where((lane == i1) | (lane >= E), -1.0, probs)
    w2 = jnp.max(p2, axis=1, keepdims=True)
    i2 = jnp.min(jnp.where(p2 == w2, lane, 999), axis=1, keepdims=True)
    tw_ref[...] = jnp.where(lane == 0, w1, jnp.where(lane == 1, w2, 0.0))
    ti_ref[...] = jnp.where(lane == 0, i1, jnp.where(lane == 1, i2, 0))


def _post_call(o, x, wo, ln2, wr_pad, wsh_pad):
    return pl.pallas_call(
        _post_body,
        grid=(NQ,),
        in_specs=[
            pl.BlockSpec((BR, D), lambda i: (i, 0)),
            pl.BlockSpec((BR, D), lambda i: (i, 0)),
            pl.BlockSpec((D, D), lambda i: (0, 0)),
            pl.BlockSpec((1, D), lambda i: (0, 0)),
            pl.BlockSpec((D, 128), lambda i: (0, 0)),
            pl.BlockSpec((D, 128), lambda i: (0, 0)),
        ],
        out_specs=[
            pl.BlockSpec((BR, D), lambda i: (i, 0)),
            pl.BlockSpec((BR, D), lambda i: (i, 0)),
            pl.BlockSpec((BR, 128), lambda i: (i, 0)),
            pl.BlockSpec((BR, 128), lambda i: (i, 0)),
            pl.BlockSpec((BR, 128), lambda i: (i, 0)),
        ],
        out_shape=[
            jax.ShapeDtypeStruct((S, D), F32),
            jax.ShapeDtypeStruct((S, D), F32),
            jax.ShapeDtypeStruct((S, 128), F32),
            jax.ShapeDtypeStruct((S, 128), jnp.int32),
            jax.ShapeDtypeStruct((S, 128), F32),
        ],
    )(o, x, wo, ln2, wr_pad, wsh_pad)


def _rank_body(ti_ref, p_ref, t_ref, carry):
    i = pl.program_id(0)

    @pl.when(i == 0)
    def _():
        carry[...] = jnp.zeros_like(carry)

    ti = ti_ref[...]
    lane = lax.broadcasted_iota(jnp.int32, (BR, 128), 1)
    c1 = (lane == ti[:, 0:1]).astype(F32)
    c2 = (lane == ti[:, 1:2]).astype(F32)
    cc = c1 + c2
    r = lax.broadcasted_iota(jnp.int32, (BR, BR), 0)
    c = lax.broadcasted_iota(jnp.int32, (BR, BR), 1)
    lower = jnp.where(r > c, 1.0, 0.0).astype(F32)
    p_ref[...] = jnp.dot(lower, cc, preferred_element_type=F32) + carry[...]
    carry[...] = carry[...] + jnp.sum(cc, axis=0, keepdims=True)
    t_ref[...] = carry[...]


def _rank_call(ti):
    return pl.pallas_call(
        _rank_body,
        grid=(NQ,),
        in_specs=[pl.BlockSpec((BR, 128), lambda i: (i, 0))],
        out_specs=[
            pl.BlockSpec((BR, 128), lambda i: (i, 0)),
            pl.BlockSpec((1, 128), lambda i: (0, 0)),
        ],
        out_shape=[
            jax.ShapeDtypeStruct((S, 128), F32),
            jax.ShapeDtypeStruct((1, 128), F32),
        ],
        scratch_shapes=[pltpu.VMEM((1, 128), F32)],
    )(ti)


def _plan_body(p_ref, ti_ref, t_ref, d_ref, be_ref):
    t = t_ref[...]
    lane_r = lax.broadcasted_iota(jnp.int32, (1, 128), 1)
    cpad = jnp.floor((t + (BLK - 1.0)) / BLK) * BLK
    base = jnp.zeros((1, 128), F32)
    cum = jnp.zeros((1, 128), F32)
    for e in range(E):
        v = jnp.sum(jnp.where(lane_r == e, cpad, 0.0), axis=1, keepdims=True)
        base = base + jnp.where(lane_r > e, v, 0.0)
        cum = cum + jnp.where(lane_r >= e, v, 0.0)
    ti = ti_ref[...]
    pb = p_ref[...] + base
    lane = lax.broadcasted_iota(jnp.int32, (BR, 128), 1)
    d1 = jnp.sum(jnp.where(lane == ti[:, 0:1], pb, 0.0), axis=1, keepdims=True)
    d2 = jnp.sum(jnp.where(lane == ti[:, 1:2], pb, 0.0), axis=1, keepdims=True)
    d_ref[...] = jnp.where(lane == 0, d1,
                           jnp.where(lane == 1, d2, 0.0)).astype(jnp.int32)
    beof = jnp.zeros((1, 128), F32)
    for e in range(E):
        ce = jnp.sum(jnp.where(lane_r == e, cum, 0.0), axis=1, keepdims=True)
        beof = beof + jnp.where(lane_r.astype(F32) * BLK >= ce, 1.0, 0.0)
    be = jnp.minimum(beof, E - 1.0)
    nb = jnp.sum(cpad, axis=1, keepdims=True) * (1.0 / BLK)
    be_ref[...] = jnp.where(lane_r == NB_LANE, nb, be).astype(jnp.int32)


def _plan_call(p, ti, t):
    return pl.pallas_call(
        _plan_body,
        grid=(NQ,),
        in_specs=[
            pl.BlockSpec((BR, 128), lambda i: (i, 0)),
            pl.BlockSpec((BR, 128), lambda i: (i, 0)),
            pl.BlockSpec((1, 128), lambda i: (0, 0)),
        ],
        out_specs=[
            pl.BlockSpec((BR, 128), lambda i: (i, 0)),
            pl.BlockSpec((1, 128), lambda i: (0, 0)),
        ],
        out_shape=[
            jax.ShapeDtypeStruct((S, 128), jnp.int32),
            jax.ShapeDtypeStruct((1, 128), jnp.int32),
        ],
    )(p, ti, t)


QS = 4
D4 = D // QS
CH = 128
NAS = 2 * S * QS
NCH = NAS // CH
NSRC = (S * QS) // CH


def _sc_dispatch(h2q, dd4):
    mesh = plsc.VectorSubcoreMesh(core_axis_name="c", subcore_axis_name="s")

    @functools.partial(
        pl.kernel,
        out_type=jax.ShapeDtypeStruct((NPAD * QS, D4), F32),
        mesh=mesh,
    )
    def k(h2_hbm, dd_hbm, xs_hbm):
        def body(h2_vmem, d_vmem):
            pltpu.sync_copy(h2_vmem, xs_hbm.at[d_vmem.at[0]])

        pltpu.emit_pipeline(
            body,
            grid=(NCH,),
            in_specs=[
                pl.BlockSpec((CH, D4), lambda i: (lax.rem(i, NSRC), 0)),
                pl.BlockSpec((1, CH), lambda i: (i, 0)),
            ],
            out_specs=[],
            core_axis_name=("c", "s"),
            dimension_semantics=(pltpu.PARALLEL,),
        )(h2_hbm, dd_hbm)

    return k(h2q, dd4)


def _sc_combine(ysq, dd4):
    mesh = plsc.VectorSubcoreMesh(core_axis_name="c", subcore_axis_name="s")

    @functools.partial(
        pl.kernel,
        out_type=jax.ShapeDtypeStruct((NAS, D4), F32),
        mesh=mesh,
    )
    def k(ys_hbm, dd_hbm, o_hbm):
        def body(d_vmem, o_vmem):
            pltpu.sync_copy(ys_hbm.at[d_vmem.at[0]], o_vmem)

        pltpu.emit_pipeline(
            body,
            grid=(NCH,),
            in_specs=[pl.BlockSpec((1, CH), lambda i: (i, 0))],
            out_specs=[pl.BlockSpec((CH, D4), lambda i: (i, 0))],
            core_axis_name=("c", "s"),
            dimension_semantics=(pltpu.PARALLEL,),
        )(dd_hbm, o_hbm)

    return k(ysq, dd4)


def _moe_body(s_ref, x_ref, wg_ref, wu_ref, wd_ref, y_ref):
    i = pl.program_id(0)

    @pl.when(i < s_ref[NB_LANE])
    def _():
        x = x_ref[...].astype(jnp.bfloat16)
        g = jnp.dot(x, wg_ref[0].astype(jnp.bfloat16),
                    preferred_element_type=F32)
        u = jnp.dot(x, wu_ref[0].astype(jnp.bfloat16),
                    preferred_element_type=F32)
        a = (g * u / (1.0 + jnp.exp(-g))).astype(jnp.bfloat16)
        y_ref[...] = jnp.dot(a, wd_ref[0].astype(jnp.bfloat16),
                             preferred_element_type=F32)


def _moe_call(s, xs, wge, wue, wde):
    grid_spec = pltpu.PrefetchScalarGridSpec(
        num_scalar_prefetch=1,
        grid=(NB,),
        in_specs=[
            pl.BlockSpec((BLK, D), lambda i, s: (i, 0)),
            pl.BlockSpec((1, D, FF), lambda i, s: (s[i], 0, 0)),
            pl.BlockSpec((1, D, FF), lambda i, s: (s[i], 0, 0)),
            pl.BlockSpec((1, FF, D), lambda i, s: (s[i], 0, 0)),
        ],
        out_specs=pl.BlockSpec((BLK, D), lambda i, s: (i, 0)),
    )
    return pl.pallas_call(
        _moe_body,
        grid_spec=grid_spec,
        out_shape=jax.ShapeDtypeStruct((NPAD, D), F32),
    )(s, xs, wge, wue, wde)


def _shared_body(h2_ref, wg_ref, wu_ref, wd_ref, page_ref):
    h2 = h2_ref[...].astype(jnp.bfloat16)
    g = jnp.dot(h2, wg_ref[...].astype(jnp.bfloat16),
                preferred_element_type=F32)
    u = jnp.dot(h2, wu_ref[...].astype(jnp.bfloat16),
                preferred_element_type=F32)
    a = (g * u / (1.0 + jnp.exp(-g))).astype(jnp.bfloat16)
    page_ref[0] = jnp.dot(a, wd_ref[...].astype(jnp.bfloat16),
                          preferred_element_type=F32)


def _shared_call(h2, wsg, wsu, wsd):
    return pl.pallas_call(
        _shared_body,
        grid=(NF, NQ),
        in_specs=[
            pl.BlockSpec((BR, D), lambda f, r: (r, 0)),
            pl.BlockSpec((D, FF), lambda f, r: (0, f)),
            pl.BlockSpec((D, FF), lambda f, r: (0, f)),
            pl.BlockSpec((FF, D), lambda f, r: (f, 0)),
        ],
        out_specs=pl.BlockSpec((1, BR, D), lambda f, r: (f, r, 0)),
        out_shape=jax.ShapeDtypeStruct((NF, S, D), F32),
    )(h2, wsg, wsu, wsd)


def _final_body(x2_ref, pages_ref, gate_ref, y0_ref, y1_ref, tw_ref, o_ref):
    tw = tw_ref[...]
    shared = pages_ref[0]
    for f in range(1, NF):
        shared = shared + pages_ref[f]
    o_ref[...] = (x2_ref[...] + gate_ref[:, 0:1] * shared
                  + tw[:, 0:1] * y0_ref[...] + tw[:, 1:2] * y1_ref[...])


def _final_call(x2, pages, gate, y01, tw):
    return pl.pallas_call(
        _final_body,
        grid=(NQ,),
        in_specs=[
            pl.BlockSpec((BR, D), lambda i: (i, 0)),
            pl.BlockSpec((NF, BR, D), lambda i: (0, i, 0)),
            pl.BlockSpec((BR, 128), lambda i: (i, 0)),
            pl.BlockSpec((BR, D), lambda i: (i, 0)),
            pl.BlockSpec((BR, D), lambda i: (i + NQ, 0)),
            pl.BlockSpec((BR, 128), lambda i: (i, 0)),
        ],
        out_specs=pl.BlockSpec((BR, D), lambda i: (i, 0)),
        out_shape=jax.ShapeDtypeStruct((S, D), F32),
    )(x2, pages, gate, y01, y01, tw)


def kernel(hidden_states, w_ln1, w_ln2, Wq, bq, Wk, bk, Wv, bv, Wo, Wr,
           Wge, Wue, Wde, Wsg, Wsu, Wsd, Wshg):
    x = hidden_states.reshape(S, D)
    pos = jnp.arange(S, dtype=F32)
    inv = 1.0 / (10000.0 ** (jnp.arange(0, HD, 2, dtype=F32) / HD))
    fr = pos[:, None] * inv[None, :]
    cos = jnp.tile(jnp.cos(fr), (1, 2 * H))
    sin = jnp.tile(jnp.sin(fr), (1, 2 * H))
    wq_p = Wq.reshape(D, H, 2, HD // 2).transpose(0, 2, 1, 3).reshape(D, D)
    wk_p = Wk.reshape(D, H, 2, HD // 2).transpose(0, 2, 1, 3).reshape(D, D)
    bq_p = bq.reshape(H, 2, HD // 2).transpose(1, 0, 2).reshape(1, D)
    bk_p = bk.reshape(H, 2, HD // 2).transpose(1, 0, 2).reshape(1, D)
    wr_pad = jnp.zeros((D, 128), F32).at[:, :E].set(Wr)
    wsh_pad = jnp.zeros((D, 128), F32).at[:, :1].set(Wshg)
    ln1 = w_ln1.reshape(1, D)
    ln2 = w_ln2.reshape(1, D)

    qf, kf, vf = _qkv_call(x, ln1, wq_p, bq_p, wk_p, bk_p,
                           Wv, bv.reshape(1, D), cos, sin)
    q = qf.reshape(S, 2, H, HD // 2).transpose(2, 0, 1, 3).reshape(H, S, HD)
    k = kf.reshape(S, 2, H, HD // 2).transpose(2, 0, 1, 3).reshape(H, S, HD)
    v = vf.reshape(S, H, HD).transpose(1, 0, 2)
    o = _attn_call(q, k, v)
    ot = o.transpose(1, 0, 2).reshape(S, D)
    x2, h2, tw, ti, gate = _post_call(ot, x, Wo, ln2, wr_pad, wsh_pad)
    p, t = _rank_call(ti)
    d12, be = _plan_call(p, ti, t)
    dd = jnp.concatenate([d12[:, 0], d12[:, 1]])
    dd4 = (dd[:, None] * QS + jnp.arange(QS, dtype=jnp.int32)[None, :])
    dd4 = dd4.reshape(NCH, CH)
    xsq = _sc_dispatch(h2.reshape(S * QS, D4), dd4)
    ys = _moe_call(be.reshape(128), xsq.reshape(NPAD, D), Wge, Wue, Wde)
    y01 = _sc_combine(ys.reshape(NPAD * QS, D4), dd4).reshape(2 * S, D)
    pages = _shared_call(h2, Wsg, Wsu, Wsd)
    out = _final_call(x2, pages, gate, y01, tw)
    return out.reshape(1, S, D)


# speedup vs baseline: 1.1367x; 1.1367x over previous
import functools

import jax
import jax.numpy as jnp
from jax import lax
from jax.experimental import pallas as pl
from jax.experimental.pallas import tpu as pltpu
from jax.experimental.pallas import tpu_sc as plsc

S, D, H, HD = 2048, 1024, 16, 64
E, FF, SFF = 8, 1408, 5632
EPS = 1e-6
F32 = jnp.float32

BR = 256
NQ = S // BR
BLK = 128
NB = (2 * S) // BLK + E
NPAD = NB * BLK
NB_LANE = 40
NF = SFF // FF
W = 32
SCALE = 1.0 / 8.0
NEG = -1e30


def _qkv_body(x_ref, ln_ref, wq_ref, bq_ref, wk_ref, bk_ref, wv_ref, bv_ref,
              cos_ref, sin_ref, q_ref, k_ref, v_ref):
    x = x_ref[...]
    var = jnp.mean(x * x, axis=1, keepdims=True)
    hn = (x * lax.rsqrt(var + EPS) * ln_ref[...]).astype(jnp.bfloat16)
    q = jnp.dot(hn, wq_ref[...].astype(jnp.bfloat16),
                preferred_element_type=F32) + bq_ref[...]
    k = jnp.dot(hn, wk_ref[...].astype(jnp.bfloat16),
                preferred_element_type=F32) + bk_ref[...]
    v = jnp.dot(hn, wv_ref[...].astype(jnp.bfloat16),
                preferred_element_type=F32) + bv_ref[...]
    cos = cos_ref[...]
    sin = sin_ref[...]
    for h in range(H):
        qh = q[:, h * HD:(h + 1) * HD]
        kh = k[:, h * HD:(h + 1) * HD]
        qrot = jnp.concatenate([-qh[:, HD // 2:], qh[:, :HD // 2]], axis=1)
        krot = jnp.concatenate([-kh[:, HD // 2:], kh[:, :HD // 2]], axis=1)
        q_ref[h] = ((qh * cos + qrot * sin) * SCALE).astype(jnp.bfloat16)
        k_ref[h] = (kh * cos + krot * sin).astype(jnp.bfloat16)
        v_ref[h] = v[:, h * HD:(h + 1) * HD].astype(jnp.bfloat16)


def _qkv_call(x, ln1, wq, bq, wk, bk, wv, bv, cos, sin):
    hsd = jax.ShapeDtypeStruct((H, S, HD), jnp.bfloat16)
    return pl.pallas_call(
        _qkv_body,
        grid=(NQ,),
        in_specs=[
            pl.BlockSpec((BR, D), lambda i: (i, 0)),
            pl.BlockSpec((1, D), lambda i: (0, 0)),
            pl.BlockSpec((D, D), lambda i: (0, 0)),
            pl.BlockSpec((1, D), lambda i: (0, 0)),
            pl.BlockSpec((D, D), lambda i: (0, 0)),
            pl.BlockSpec((1, D), lambda i: (0, 0)),
            pl.BlockSpec((D, D), lambda i: (0, 0)),
            pl.BlockSpec((1, D), lambda i: (0, 0)),
            pl.BlockSpec((BR, HD), lambda i: (i, 0)),
            pl.BlockSpec((BR, HD), lambda i: (i, 0)),
        ],
        out_specs=[
            pl.BlockSpec((H, BR, HD), lambda i: (0, i, 0)),
            pl.BlockSpec((H, BR, HD), lambda i: (0, i, 0)),
            pl.BlockSpec((H, BR, HD), lambda i: (0, i, 0)),
        ],
        out_shape=[hsd, hsd, hsd],
    )(x, ln1, wq, bq, wk, bk, wv, bv, cos, sin)


BQA = 512
NQA = S // BQA
TRI = NQA * (NQA + 1) // 2


def _attn_body(sq_ref, sk_ref, sd_ref, q_ref, k_ref, v_ref, mask_ref, o_ref,
               l_sc, a_sc):
    t = pl.program_id(1)
    iq = sq_ref[t]
    jk = sk_ref[t]

    @pl.when(jk == 0)
    def _():
        l_sc[...] = jnp.zeros((BQA, 128), F32)
        a_sc[...] = jnp.zeros((BQA, HD), F32)

    q = q_ref[0]
    kk = k_ref[0]
    s = lax.dot_general(q, kk, (((1,), (1,)), ((), ())),
                        preferred_element_type=F32)
    p = jnp.exp(jnp.minimum(s + mask_ref[0], 60.0))
    l_new = l_sc[:, 0:1] + jnp.sum(p, axis=1, keepdims=True)
    a_new = a_sc[...] + jnp.dot(
        p.astype(jnp.bfloat16), v_ref[0], preferred_element_type=F32)
    l_sc[...] = jnp.broadcast_to(l_new, (BQA, 128))
    a_sc[...] = a_new

    @pl.when(jk == iq)
    def _():
        o_ref[0] = a_new / l_new


def _attn_call(q, k, v):
    sq = []
    sk = []
    for i in range(NQA):
        for j in range(i + 1):
            sq.append(i)
            sk.append(j)
    diag = [int(a == b) for a, b in zip(sq, sk)]
    sq = jnp.asarray(sq, jnp.int32)
    sk = jnp.asarray(sk, jnp.int32)
    sd = jnp.asarray(diag, jnp.int32)
    r = jnp.arange(BQA, dtype=jnp.int32)
    causal = jnp.where(r[:, None] >= r[None, :], 0.0, NEG).astype(F32)
    masks = jnp.stack([jnp.zeros((BQA, BQA), F32), causal])
    grid_spec = pltpu.PrefetchScalarGridSpec(
        num_scalar_prefetch=3,
        grid=(H, TRI),
        in_specs=[
            pl.BlockSpec((1, BQA, HD), lambda h, t, sq, sk, sd: (h, sq[t], 0)),
            pl.BlockSpec((1, BQA, HD), lambda h, t, sq, sk, sd: (h, sk[t], 0)),
            pl.BlockSpec((1, BQA, HD), lambda h, t, sq, sk, sd: (h, sk[t], 0)),
            pl.BlockSpec((1, BQA, BQA), lambda h, t, sq, sk, sd: (sd[t], 0, 0)),
        ],
        out_specs=pl.BlockSpec(
            (1, BQA, HD), lambda h, t, sq, sk, sd: (h, sq[t], 0)),
        scratch_shapes=[
            pltpu.VMEM((BQA, 128), F32),
            pltpu.VMEM((BQA, HD), F32),
        ],
    )
    return pl.pallas_call(
        _attn_body,
        grid_spec=grid_spec,
        out_shape=jax.ShapeDtypeStruct((H, S, HD), F32),
    )(sq, sk, sd, q, k, v, masks)


def _post_body(o_ref, x_ref, wo_ref, ln2_ref, wr_ref, wsh_ref,
               x2_ref, h2_ref, tw_ref, ti_ref, gate_ref):
    o = jnp.concatenate([o_ref[h] for h in range(H)], axis=1)
    x2 = x_ref[...] + jnp.dot(o.astype(jnp.bfloat16),
                              wo_ref[...].astype(jnp.bfloat16),
                              preferred_element_type=F32)
    x2_ref[...] = x2
    var = jnp.mean(x2 * x2, axis=1, keepdims=True)
    h2 = x2 * lax.rsqrt(var + EPS) * ln2_ref[...]
    h2_ref[...] = h2
    lg = jnp.dot(h2, wsh_ref[...], preferred_element_type=F32)
    gate_ref[...] = 1.0 / (1.0 + jnp.exp(-lg))
    logits = jnp.dot(h2, wr_ref[...], preferred_element_type=F32)
    lane = lax.broadcasted_iota(jnp.int32, (BR, 128), 1)
    logits = jnp.where(lane < E, logits, NEG)
    mx = jnp.max(logits, axis=1, keepdims=True)
    ex = jnp.exp(logits - mx)
    probs = ex / jnp.sum(ex, axis=1, keepdims=True)
    w1 = jnp.max(probs, axis=1, keepdims=True)
    i1 = jnp.min(jnp.where(probs == w1, lane, 999), axis=1, keepdims=True)
    p2 = jnp.where((lane == i1) | (lane >= E), -1.0, probs)
    w2 = jnp.max(p2, axis=1, keepdims=True)
    i2 = jnp.min(jnp.where(p2 == w2, lane, 999), axis=1, keepdims=True)
    tw_ref[...] = jnp.where(lane == 0, w1, jnp.where(lane == 1, w2, 0.0))
    ti_ref[...] = jnp.where(lane == 0, i1, jnp.where(lane == 1, i2, 0))


def _post_call(o, x, wo, ln2, wr_pad, wsh_pad):
    return pl.pallas_call(
        _post_body,
        grid=(NQ,),
        in_specs=[
            pl.BlockSpec((H, BR, HD), lambda i: (0, i, 0)),
            pl.BlockSpec((BR, D), lambda i: (i, 0)),
            pl.BlockSpec((D, D), lambda i: (0, 0)),
            pl.BlockSpec((1, D), lambda i: (0, 0)),
            pl.BlockSpec((D, 128), lambda i: (0, 0)),
            pl.BlockSpec((D, 128), lambda i: (0, 0)),
        ],
        out_specs=[
            pl.BlockSpec((BR, D), lambda i: (i, 0)),
            pl.BlockSpec((BR, D), lambda i: (i, 0)),
            pl.BlockSpec((BR, 128), lambda i: (i, 0)),
            pl.BlockSpec((BR, 128), lambda i: (i, 0)),
            pl.BlockSpec((BR, 128), lambda i: (i, 0)),
        ],
        out_shape=[
            jax.ShapeDtypeStruct((S, D), F32),
            jax.ShapeDtypeStruct((S, D), F32),
            jax.ShapeDtypeStruct((S, 128), F32),
            jax.ShapeDtypeStruct((S, 128), jnp.int32),
            jax.ShapeDtypeStruct((S, 128), F32),
        ],
    )(o, x, wo, ln2, wr_pad, wsh_pad)


def _rank_body(ti_ref, p_ref, t_ref, carry):
    i = pl.program_id(0)

    @pl.when(i == 0)
    def _():
        carry[...] = jnp.zeros_like(carry)

    ti = ti_ref[...]
    lane = lax.broadcasted_iota(jnp.int32, (BR, 128), 1)
    c1 = (lane == ti[:, 0:1]).astype(F32)
    c2 = (lane == ti[:, 1:2]).astype(F32)
    cc = c1 + c2
    r = lax.broadcasted_iota(jnp.int32, (BR, BR), 0)
    c = lax.broadcasted_iota(jnp.int32, (BR, BR), 1)
    lower = jnp.where(r > c, 1.0, 0.0).astype(F32)
    p_ref[...] = jnp.dot(lower, cc, preferred_element_type=F32) + carry[...]
    carry[...] = carry[...] + jnp.sum(cc, axis=0, keepdims=True)
    t_ref[...] = carry[...]


def _rank_call(ti):
    return pl.pallas_call(
        _rank_body,
        grid=(NQ,),
        in_specs=[pl.BlockSpec((BR, 128), lambda i: (i, 0))],
        out_specs=[
            pl.BlockSpec((BR, 128), lambda i: (i, 0)),
            pl.BlockSpec((1, 128), lambda i: (0, 0)),
        ],
        out_shape=[
            jax.ShapeDtypeStruct((S, 128), F32),
            jax.ShapeDtypeStruct((1, 128), F32),
        ],
        scratch_shapes=[pltpu.VMEM((1, 128), F32)],
    )(ti)


def _plan_body(p_ref, ti_ref, t_ref, d_ref, be_ref):
    t = t_ref[...]
    lane_r = lax.broadcasted_iota(jnp.int32, (1, 128), 1)
    cpad = jnp.floor((t + (BLK - 1.0)) / BLK) * BLK
    base = jnp.zeros((1, 128), F32)
    cum = jnp.zeros((1, 128), F32)
    for e in range(E):
        v = jnp.sum(jnp.where(lane_r == e, cpad, 0.0), axis=1, keepdims=True)
        base = base + jnp.where(lane_r > e, v, 0.0)
        cum = cum + jnp.where(lane_r >= e, v, 0.0)
    ti = ti_ref[...]
    pb = p_ref[...] + base
    lane = lax.broadcasted_iota(jnp.int32, (BR, 128), 1)
    d1 = jnp.sum(jnp.where(lane == ti[:, 0:1], pb, 0.0), axis=1, keepdims=True)
    d2 = jnp.sum(jnp.where(lane == ti[:, 1:2], pb, 0.0), axis=1, keepdims=True)
    d_ref[...] = jnp.where(lane == 0, d1,
                           jnp.where(lane == 1, d2, 0.0)).astype(jnp.int32)
    beof = jnp.zeros((1, 128), F32)
    for e in range(E):
        ce = jnp.sum(jnp.where(lane_r == e, cum, 0.0), axis=1, keepdims=True)
        beof = beof + jnp.where(lane_r.astype(F32) * BLK >= ce, 1.0, 0.0)
    be = jnp.minimum(beof, E - 1.0)
    nb = jnp.sum(cpad, axis=1, keepdims=True) * (1.0 / BLK)
    be_ref[...] = jnp.where(lane_r == NB_LANE, nb, be).astype(jnp.int32)


def _plan_call(p, ti, t):
    return pl.pallas_call(
        _plan_body,
        grid=(NQ,),
        in_specs=[
            pl.BlockSpec((BR, 128), lambda i: (i, 0)),
            pl.BlockSpec((BR, 128), lambda i: (i, 0)),
            pl.BlockSpec((1, 128), lambda i: (0, 0)),
        ],
        out_specs=[
            pl.BlockSpec((BR, 128), lambda i: (i, 0)),
            pl.BlockSpec((1, 128), lambda i: (0, 0)),
        ],
        out_shape=[
            jax.ShapeDtypeStruct((S, 128), jnp.int32),
            jax.ShapeDtypeStruct((1, 128), jnp.int32),
        ],
    )(p, ti, t)


QS = 4
D4 = D // QS
CH = 128
NAS = 2 * S * QS
NCH = NAS // CH
NSRC = (S * QS) // CH


def _sc_dispatch(h2q, dd4):
    mesh = plsc.VectorSubcoreMesh(core_axis_name="c", subcore_axis_name="s")

    @functools.partial(
        pl.kernel,
        out_type=jax.ShapeDtypeStruct((NPAD * QS, D4), F32),
        mesh=mesh,
    )
    def k(h2_hbm, dd_hbm, xs_hbm):
        def body(h2_vmem, d_vmem):
            pltpu.sync_copy(h2_vmem, xs_hbm.at[d_vmem.at[0]])

        pltpu.emit_pipeline(
            body,
            grid=(NCH,),
            in_specs=[
                pl.BlockSpec((CH, D4), lambda i: (lax.rem(i, NSRC), 0)),
                pl.BlockSpec((1, CH), lambda i: (i, 0)),
            ],
            out_specs=[],
            core_axis_name=("c", "s"),
            dimension_semantics=(pltpu.PARALLEL,),
        )(h2_hbm, dd_hbm)

    return k(h2q, dd4)


def _sc_combine(ysq, dd4):
    mesh = plsc.VectorSubcoreMesh(core_axis_name="c", subcore_axis_name="s")

    @functools.partial(
        pl.kernel,
        out_type=jax.ShapeDtypeStruct((NAS, D4), F32),
        mesh=mesh,
    )
    def k(ys_hbm, dd_hbm, o_hbm):
        def body(d_vmem, o_vmem):
            pltpu.sync_copy(ys_hbm.at[d_vmem.at[0]], o_vmem)

        pltpu.emit_pipeline(
            body,
            grid=(NCH,),
            in_specs=[pl.BlockSpec((1, CH), lambda i: (i, 0))],
            out_specs=[pl.BlockSpec((CH, D4), lambda i: (i, 0))],
            core_axis_name=("c", "s"),
            dimension_semantics=(pltpu.PARALLEL,),
        )(dd_hbm, o_hbm)

    return k(ysq, dd4)


def _moe_body(s_ref, x_ref, wg_ref, wu_ref, wd_ref, y_ref):
    i = pl.program_id(0)

    @pl.when(i < s_ref[NB_LANE])
    def _():
        x = x_ref[...].astype(jnp.bfloat16)
        g = jnp.dot(x, wg_ref[0].astype(jnp.bfloat16),
                    preferred_element_type=F32)
        u = jnp.dot(x, wu_ref[0].astype(jnp.bfloat16),
                    preferred_element_type=F32)
        a = (g * u / (1.0 + jnp.exp(-g))).astype(jnp.bfloat16)
        y_ref[...] = jnp.dot(a, wd_ref[0].astype(jnp.bfloat16),
                             preferred_element_type=F32)


def _moe_call(s, xs, wge, wue, wde):
    grid_spec = pltpu.PrefetchScalarGridSpec(
        num_scalar_prefetch=1,
        grid=(NB,),
        in_specs=[
            pl.BlockSpec((BLK, D), lambda i, s: (i, 0)),
            pl.BlockSpec((1, D, FF), lambda i, s: (s[i], 0, 0)),
            pl.BlockSpec((1, D, FF), lambda i, s: (s[i], 0, 0)),
            pl.BlockSpec((1, FF, D), lambda i, s: (s[i], 0, 0)),
        ],
        out_specs=pl.BlockSpec((BLK, D), lambda i, s: (i, 0)),
    )
    return pl.pallas_call(
        _moe_body,
        grid_spec=grid_spec,
        out_shape=jax.ShapeDtypeStruct((NPAD, D), F32),
    )(s, xs, wge, wue, wde)


def _shared_body(h2_ref, wg_ref, wu_ref, wd_ref, page_ref):
    h2 = h2_ref[...].astype(jnp.bfloat16)
    g = jnp.dot(h2, wg_ref[...].astype(jnp.bfloat16),
                preferred_element_type=F32)
    u = jnp.dot(h2, wu_ref[...].astype(jnp.bfloat16),
                preferred_element_type=F32)
    a = (g * u / (1.0 + jnp.exp(-g))).astype(jnp.bfloat16)
    page_ref[0] = jnp.dot(a, wd_ref[...].astype(jnp.bfloat16),
                          preferred_element_type=F32)


def _shared_call(h2, wsg, wsu, wsd):
    return pl.pallas_call(
        _shared_body,
        grid=(NF, NQ),
        in_specs=[
            pl.BlockSpec((BR, D), lambda f, r: (r, 0)),
            pl.BlockSpec((D, FF), lambda f, r: (0, f)),
            pl.BlockSpec((D, FF), lambda f, r: (0, f)),
            pl.BlockSpec((FF, D), lambda f, r: (f, 0)),
        ],
        out_specs=pl.BlockSpec((1, BR, D), lambda f, r: (f, r, 0)),
        out_shape=jax.ShapeDtypeStruct((NF, S, D), F32),
    )(h2, wsg, wsu, wsd)


def _final_body(x2_ref, pages_ref, gate_ref, y0_ref, y1_ref, tw_ref, o_ref):
    tw = tw_ref[...]
    shared = pages_ref[0]
    for f in range(1, NF):
        shared = shared + pages_ref[f]
    o_ref[...] = (x2_ref[...] + gate_ref[:, 0:1] * shared
                  + tw[:, 0:1] * y0_ref[...] + tw[:, 1:2] * y1_ref[...])


def _final_call(x2, pages, gate, y01, tw):
    return pl.pallas_call(
        _final_body,
        grid=(NQ,),
        in_specs=[
            pl.BlockSpec((BR, D), lambda i: (i, 0)),
            pl.BlockSpec((NF, BR, D), lambda i: (0, i, 0)),
            pl.BlockSpec((BR, 128), lambda i: (i, 0)),
            pl.BlockSpec((BR, D), lambda i: (i, 0)),
            pl.BlockSpec((BR, D), lambda i: (i + NQ, 0)),
            pl.BlockSpec((BR, 128), lambda i: (i, 0)),
        ],
        out_specs=pl.BlockSpec((BR, D), lambda i: (i, 0)),
        out_shape=jax.ShapeDtypeStruct((S, D), F32),
    )(x2, pages, gate, y01, y01, tw)


def kernel(hidden_states, w_ln1, w_ln2, Wq, bq, Wk, bk, Wv, bv, Wo, Wr,
           Wge, Wue, Wde, Wsg, Wsu, Wsd, Wshg):
    x = hidden_states.reshape(S, D)
    pos = jnp.arange(S, dtype=F32)
    inv = 1.0 / (10000.0 ** (jnp.arange(0, HD, 2, dtype=F32) / HD))
    fr = pos[:, None] * inv[None, :]
    cos = jnp.concatenate([jnp.cos(fr), jnp.cos(fr)], axis=-1)
    sin = jnp.concatenate([jnp.sin(fr), jnp.sin(fr)], axis=-1)
    wr_pad = jnp.zeros((D, 128), F32).at[:, :E].set(Wr)
    wsh_pad = jnp.zeros((D, 128), F32).at[:, :1].set(Wshg)
    ln1 = w_ln1.reshape(1, D)
    ln2 = w_ln2.reshape(1, D)

    q, k, v = _qkv_call(x, ln1, Wq, bq.reshape(1, D), Wk, bk.reshape(1, D),
                        Wv, bv.reshape(1, D), cos, sin)
    o = _attn_call(q, k, v)
    x2, h2, tw, ti, gate = _post_call(o, x, Wo, ln2, wr_pad, wsh_pad)
    p, t = _rank_call(ti)
    d12, be = _plan_call(p, ti, t)
    dd = jnp.concatenate([d12[:, 0], d12[:, 1]])
    dd4 = (dd[:, None] * QS + jnp.arange(QS, dtype=jnp.int32)[None, :])
    dd4 = dd4.reshape(NCH, CH)
    xsq = _sc_dispatch(h2.reshape(S * QS, D4), dd4)
    ys = _moe_call(be.reshape(128), xsq.reshape(NPAD, D), Wge, Wue, Wde)
    y01 = _sc_combine(ys.reshape(NPAD * QS, D4), dd4).reshape(2 * S, D)
    pages = _shared_call(h2, Wsg, Wsu, Wsd)
    out = _final_call(x2, pages, gate, y01, tw)
    return out.reshape(1, S, D)
